# Initial kernel scaffold; baseline (speedup 1.0000x reference)
import jax
import jax.numpy as jnp
from jax.experimental import pallas as pl


def kernel(edge_index, h, e, A1_W, A1_b, A2_W, A2_b, A3_W, A3_b, B1_W, B1_b, B2_W, B2_b, B3_W, B3_b, bn_h_gamma, bn_h_beta, bn_e_gamma, bn_e_beta):
    raise NotImplementedError("write your pallas kernel here")


# speedup vs baseline: 2.3415x; 2.3415x over previous
import functools

import jax
import jax.numpy as jnp
from jax import lax
from jax.experimental import pallas as pl
from jax.experimental.pallas import tpu as pltpu
from jax.experimental.pallas import tpu_sc as plsc

F32 = jnp.float32
NC = 2
NS = 16
NW = NC * NS
K = 80
DC = 32

_mesh = plsc.VectorSubcoreMesh(core_axis_name="c", subcore_axis_name="s")


def _node_matmuls(h, wts, bs, block=2000):
    N, D = h.shape
    nb = N // block

    def body(h_ref, w_ref, b_ref, o1, o2, o3, o4, o5):
        hh = h_ref[...]
        outs = (o1, o2, o3, o4, o5)
        for i in range(5):
            outs[i][...] = jnp.dot(hh, w_ref[i], precision=lax.Precision.HIGHEST,
                                   preferred_element_type=F32) + b_ref[i]

    out = jax.ShapeDtypeStruct((N, D), F32)
    blk = pl.BlockSpec((block, D), lambda i: (i, 0))
    return pl.pallas_call(
        body,
        grid=(nb,),
        in_specs=[blk,
                  pl.BlockSpec((5, D, D), lambda i: (0, 0, 0)),
                  pl.BlockSpec((5, 1, D), lambda i: (0, 0, 0))],
        out_specs=[blk] * 5,
        out_shape=[out] * 5,
    )(h, jnp.stack(wts), jnp.stack(bs)[:, None, :])


def _sc_gather(ei4, B1h, B2h, A2h, A3h, E):
    N, D = B1h.shape
    nchunk = E // (NW * K)
    fo = jax.ShapeDtypeStruct((E, D), F32)

    @functools.partial(
        pl.kernel, mesh=_mesh,
        out_type=[fo, fo, fo, fo],
        scratch_types=[
            pltpu.VMEM((K,), jnp.int32),
            pltpu.VMEM((K,), jnp.int32),
            pltpu.VMEM((K, D), F32),
            pltpu.VMEM((K, D), F32),
            pltpu.VMEM((K, D), F32),
            pltpu.VMEM((K, D), F32),
            pltpu.SemaphoreType.DMA,
            pltpu.SemaphoreType.DMA,
            pltpu.SemaphoreType.DMA,
            pltpu.SemaphoreType.DMA,
        ],
    )
    def k(ei_hbm, b1_hbm, b2_hbm, a2_hbm, a3_hbm,
          gb1_hbm, gb2_hbm, ga2_hbm, ga3_hbm,
          idxs, idxd, buf1, buf2, buf3, buf4, s1, s2, s3, s4):
        wid = lax.axis_index("s") * NC + lax.axis_index("c")

        @pl.loop(0, nchunk)
        def _(i):
            off = (wid * nchunk + i) * K
            pltpu.sync_copy(ei_hbm.at[0, wid, i], idxs)
            pltpu.sync_copy(ei_hbm.at[1, wid, i], idxd)
            c1 = pltpu.async_copy(b1_hbm.at[idxs], buf1, s1)
            c2 = pltpu.async_copy(b2_hbm.at[idxd], buf2, s2)
            c3 = pltpu.async_copy(a2_hbm.at[idxs], buf3, s3)
            c4 = pltpu.async_copy(a3_hbm.at[idxd], buf4, s4)
            c1.wait()
            pltpu.sync_copy(buf1, gb1_hbm.at[pl.ds(off, K)])
            c2.wait()
            pltpu.sync_copy(buf2, gb2_hbm.at[pl.ds(off, K)])
            c3.wait()
            pltpu.sync_copy(buf3, ga2_hbm.at[pl.ds(off, K)])
            c4.wait()
            pltpu.sync_copy(buf4, ga3_hbm.at[pl.ds(off, K)])

    return k(ei4, B1h, B2h, A2h, A3h)


def _edge_t_stats(e, gb1, gb2, wt, b, block):
    E, D = e.shape
    nb = E // block

    def body(e_ref, g1_ref, g2_ref, w_ref, b_ref, t_ref, st_ref, acc):
        i = pl.program_id(0)

        @pl.when(i == 0)
        def _():
            acc[...] = jnp.zeros_like(acc)

        t = (jnp.dot(e_ref[...], w_ref[...], precision=lax.Precision.HIGHEST,
                     preferred_element_type=F32)
             + b_ref[...] + g1_ref[...] + g2_ref[...])
        t_ref[...] = t
        t3 = t.reshape(block // 8, 8, D)
        acc[0] += jnp.sum(t3, axis=0)
        acc[1] += jnp.sum(t3 * t3, axis=0)

        @pl.when(i == nb - 1)
        def _():
            st_ref[...] = acc[...]

    blk = pl.BlockSpec((block, D), lambda i: (i, 0))
    return pl.pallas_call(
        body,
        grid=(nb,),
        in_specs=[blk, blk, blk,
                  pl.BlockSpec((D, D), lambda i: (0, 0)),
                  pl.BlockSpec((1, D), lambda i: (0, 0))],
        out_specs=[blk, pl.BlockSpec((2, 8, D), lambda i: (0, 0, 0))],
        out_shape=[jax.ShapeDtypeStruct((E, D), F32),
                   jax.ShapeDtypeStruct((2, 8, D), F32)],
        scratch_shapes=[pltpu.VMEM((2, 8, D), F32)],
    )(e, gb1, gb2, wt, b[None, :])


def _edge_sigma(t, e, ga2, ga3, stats, gamma, beta, block):
    E, D = e.shape
    nb = E // block

    def body(t_ref, e_ref, a2_ref, a3_ref, st_ref, g_ref, bt_ref,
             sg_ref, u2_ref, u3_ref):
        mean = jnp.sum(st_ref[0], axis=0) / E
        var = jnp.sum(st_ref[1], axis=0) / E - mean * mean
        scale = g_ref[0] * lax.rsqrt(var + 1e-5)
        shift = bt_ref[0] - mean * scale
        bn = t_ref[...] * scale + shift
        ee = jnp.maximum(bn, 0.0) + e_ref[...]
        sg = jax.nn.sigmoid(ee)
        sg_ref[...] = sg
        u2_ref[...] = sg * a2_ref[...]
        u3_ref[...] = sg * a3_ref[...]

    blk = pl.BlockSpec((block, D), lambda i: (i, 0))
    small = pl.BlockSpec((1, D), lambda i: (0, 0))
    out = jax.ShapeDtypeStruct((E, D), F32)
    return pl.pallas_call(
        body,
        grid=(nb,),
        in_specs=[blk, blk, blk, blk,
                  pl.BlockSpec((2, 8, D), lambda i: (0, 0, 0)), small, small],
        out_specs=[blk, blk, blk],
        out_shape=[out, out, out],
    )(t, e, ga2, ga3, stats, gamma[None, :], beta[None, :])


def _sc_scatter(ei4, sigma, u2, u3, zrows, N):
    E, D = sigma.shape
    nchunk = E // (NW * K)
    ncc = D // DC
    nrow = N // NS
    po = jax.ShapeDtypeStruct((NC, N, D), F32)

    @functools.partial(
        pl.kernel, mesh=_mesh,
        out_type=[po, po, po, po],
        compiler_params=pltpu.CompilerParams(use_tc_tiling_on_sc=False),
        scratch_types=[
            pltpu.VMEM((nchunk, K), jnp.int32),
            pltpu.VMEM((nchunk, K), jnp.int32),
            pltpu.VMEM((K, DC), F32),
            pltpu.VMEM((K, DC), F32),
            pltpu.VMEM((K, DC), F32),
            pltpu.VMEM_SHARED((N, DC), F32),
            pltpu.VMEM_SHARED((N, DC), F32),
            pltpu.VMEM_SHARED((N, DC), F32),
            pltpu.VMEM_SHARED((N, DC), F32),
        ],
    )
    def k(ei_hbm, sg_hbm, u2_hbm, u3_hbm, z_hbm,
          oaF, odF, oaB, odB,
          idxs2, idxd2, sb, u2b, u3b, aF, dF, aB, dB):
        cid = lax.axis_index("c")
        sid = lax.axis_index("s")
        wid = sid * NC + cid
        r0 = sid * nrow
        pltpu.sync_copy(ei_hbm.at[0, wid], idxs2)
        pltpu.sync_copy(ei_hbm.at[1, wid], idxd2)
        for c in range(ncc):
            cs = c * DC
            pltpu.sync_copy(z_hbm, aF.at[pl.ds(r0, nrow)])
            pltpu.sync_copy(z_hbm, dF.at[pl.ds(r0, nrow)])
            pltpu.sync_copy(z_hbm, aB.at[pl.ds(r0, nrow)])
            pltpu.sync_copy(z_hbm, dB.at[pl.ds(r0, nrow)])
            plsc.subcore_barrier()

            @pl.loop(0, nchunk)
            def _(i):
                off = (wid * nchunk + i) * K
                pltpu.sync_copy(sg_hbm.at[pl.ds(off, K), pl.ds(cs, DC)], sb)
                pltpu.sync_copy(u2_hbm.at[pl.ds(off, K), pl.ds(cs, DC)], u2b)
                pltpu.sync_copy(u3_hbm.at[pl.ds(off, K), pl.ds(cs, DC)], u3b)
                pltpu.sync_copy(u2b, aF.at[idxd2.at[i]], add=True)
                pltpu.sync_copy(sb, dF.at[idxd2.at[i]], add=True)
                pltpu.sync_copy(u3b, aB.at[idxs2.at[i]], add=True)
                pltpu.sync_copy(sb, dB.at[idxs2.at[i]], add=True)

            plsc.subcore_barrier()
            pltpu.sync_copy(aF.at[pl.ds(r0, nrow)],
                            oaF.at[cid, pl.ds(r0, nrow), pl.ds(cs, DC)])
            pltpu.sync_copy(dF.at[pl.ds(r0, nrow)],
                            odF.at[cid, pl.ds(r0, nrow), pl.ds(cs, DC)])
            pltpu.sync_copy(aB.at[pl.ds(r0, nrow)],
                            oaB.at[cid, pl.ds(r0, nrow), pl.ds(cs, DC)])
            pltpu.sync_copy(dB.at[pl.ds(r0, nrow)],
                            odB.at[cid, pl.ds(r0, nrow), pl.ds(cs, DC)])
            plsc.subcore_barrier()

    return k(ei4, sigma, u2, u3, zrows)


def _final(A1h, h, aF, dF, aB, dB, gamma, beta, block):
    N, D = h.shape
    nb = N // block

    def body(a1_ref, h_ref, aF_ref, dF_ref, aB_ref, dB_ref, g_ref, bt_ref,
             o_ref, acc):
        p = pl.program_id(0)
        j = pl.program_id(1)
        hf = (aF_ref[0] + aF_ref[1]) / (dF_ref[0] + dF_ref[1] + 1e-6)
        hb = (aB_ref[0] + aB_ref[1]) / (dB_ref[0] + dB_ref[1] + 1e-6)
        hp = a1_ref[...] + hf + hb

        @pl.when(p == 0)
        def _():
            @pl.when(j == 0)
            def _():
                acc[...] = jnp.zeros_like(acc)

            h3 = hp.reshape(block // 8, 8, D)
            acc[0] += jnp.sum(h3, axis=0)
            acc[1] += jnp.sum(h3 * h3, axis=0)

        @pl.when(p == 1)
        def _():
            mean = jnp.sum(acc[0], axis=0) / N
            var = jnp.sum(acc[1], axis=0) / N - mean * mean
            scale = g_ref[0] * lax.rsqrt(var + 1e-5)
            shift = bt_ref[0] - mean * scale
            o_ref[...] = jnp.maximum(hp * scale + shift, 0.0) + h_ref[...]

    blk = pl.BlockSpec((block, D), lambda p, j: (j, 0))
    pblk = pl.BlockSpec((2, block, D), lambda p, j: (0, j, 0))
    small = pl.BlockSpec((1, D), lambda p, j: (0, 0))
    return pl.pallas_call(
        body,
        grid=(2, nb),
        in_specs=[blk, blk, pblk, pblk, pblk, pblk, small, small],
        out_specs=blk,
        out_shape=jax.ShapeDtypeStruct((N, D), F32),
        scratch_shapes=[pltpu.VMEM((2, 8, D), F32)],
    )(A1h, h, aF, dF, aB, dB, gamma[None, :], beta[None, :])


def kernel(edge_index, h, e, A1_W, A1_b, A2_W, A2_b, A3_W, A3_b,
           B1_W, B1_b, B2_W, B2_b, B3_W, B3_b,
           bn_h_gamma, bn_h_beta, bn_e_gamma, bn_e_beta):
    N, D = h.shape
    E = e.shape[0]
    nchunk = E // (NW * K)

    A1h, A2h, A3h, B1h, B2h = _node_matmuls(
        h,
        (A1_W.T, A2_W.T, A3_W.T, B1_W.T, B2_W.T),
        (A1_b, A2_b, A3_b, B1_b, B2_b))

    ei4 = edge_index.reshape(2, NW, nchunk, K)
    gb1, gb2, ga2, ga3 = _sc_gather(ei4, B1h, B2h, A2h, A3h, E)

    t, stats = _edge_t_stats(e, gb1, gb2, B3_W.T, B3_b, 2000)
    sigma, u2, u3 = _edge_sigma(t, e, ga2, ga3, stats,
                                bn_e_gamma, bn_e_beta, 2000)

    zrows = jnp.zeros((N // NS, DC), F32)
    aF, dF, aB, dB = _sc_scatter(ei4, sigma, u2, u3, zrows, N)

    return _final(A1h, h, aF, dF, aB, dB, bn_h_gamma, bn_h_beta, 2000)


# speedup vs baseline: 2.8415x; 1.2135x over previous
import functools

import jax
import jax.numpy as jnp
from jax import lax
from jax.experimental import pallas as pl
from jax.experimental.pallas import tpu as pltpu
from jax.experimental.pallas import tpu_sc as plsc

F32 = jnp.float32
NC = 2
NS = 16
NW = NC * NS
K = 80
DC = 32

_mesh = plsc.VectorSubcoreMesh(core_axis_name="c", subcore_axis_name="s")


def _node_matmuls(h, wts, bs, block=2000):
    N, D = h.shape
    nb = N // block

    def body(h_ref, w_ref, b_ref, oa1, ot1, ot2):
        hh = h_ref[...]
        m = [jnp.dot(hh, w_ref[i], precision=lax.Precision.HIGHEST,
                     preferred_element_type=F32) + b_ref[i] for i in range(5)]
        oa1[...] = m[0]
        ot1[...] = jnp.concatenate([m[1], m[2]], axis=1)
        ot2[...] = jnp.concatenate([m[3], m[4]], axis=1)

    blk = pl.BlockSpec((block, D), lambda i: (i, 0))
    blk2 = pl.BlockSpec((block, 2 * D), lambda i: (i, 0))
    return pl.pallas_call(
        body,
        grid=(nb,),
        in_specs=[blk,
                  pl.BlockSpec((5, D, D), lambda i: (0, 0, 0)),
                  pl.BlockSpec((5, 1, D), lambda i: (0, 0, 0))],
        out_specs=[blk, blk2, blk2],
        out_shape=[jax.ShapeDtypeStruct((N, D), F32),
                   jax.ShapeDtypeStruct((N, 2 * D), F32),
                   jax.ShapeDtypeStruct((N, 2 * D), F32)],
    )(h, jnp.stack(wts), jnp.stack(bs)[:, None, :])


def _sc_gather(ei4, T1, T2, E):
    N, D2 = T1.shape
    nchunk = E // (NW * K)
    npair = (nchunk - 1) // 2
    fo = jax.ShapeDtypeStruct((E, D2), F32)

    @functools.partial(
        pl.kernel, mesh=_mesh,
        out_type=[fo, fo],
        scratch_types=[
            pltpu.VMEM((nchunk, K), jnp.int32),
            pltpu.VMEM((nchunk, K), jnp.int32),
            pltpu.VMEM((K, D2), F32),
            pltpu.VMEM((K, D2), F32),
            pltpu.VMEM((K, D2), F32),
            pltpu.VMEM((K, D2), F32),
            pltpu.SemaphoreType.DMA,
            pltpu.SemaphoreType.DMA,
            pltpu.SemaphoreType.DMA,
        ],
    )
    def k(ei_hbm, t1_hbm, t2_hbm, g1_hbm, g2_hbm,
          idxs2, idxd2, a0, b0, a1, b1, sg0, sg1, sw):
        wid = lax.axis_index("s") * NC + lax.axis_index("c")
        base = wid * nchunk
        pltpu.sync_copy(ei_hbm.at[0, wid], idxs2)
        pltpu.sync_copy(ei_hbm.at[1, wid], idxd2)

        def gath(i, bufa, bufb, sem):
            pltpu.async_copy(t1_hbm.at[idxs2.at[i]], bufa, sem)
            pltpu.async_copy(t2_hbm.at[idxd2.at[i]], bufb, sem)

        def wait_g(bufa, bufb, sem):
            pltpu.make_async_copy(g1_hbm.at[pl.ds(0, K)], bufa, sem).wait()
            pltpu.make_async_copy(g1_hbm.at[pl.ds(0, K)], bufb, sem).wait()

        def wout(i, bufa, bufb):
            off = (base + i) * K
            c1 = pltpu.async_copy(bufa, g1_hbm.at[pl.ds(off, K)], sw)
            c2 = pltpu.async_copy(bufb, g2_hbm.at[pl.ds(off, K)], sw)
            c1.wait()
            c2.wait()

        gath(0, a0, b0, sg0)
        gath(1, a1, b1, sg1)

        @pl.loop(0, npair)
        def _(j):
            i = j * 2
            wait_g(a0, b0, sg0)
            wout(i, a0, b0)
            gath(i + 2, a0, b0, sg0)
            wait_g(a1, b1, sg1)
            wout(i + 1, a1, b1)

            @pl.when(j < npair - 1)
            def _():
                gath(i + 3, a1, b1, sg1)

        wait_g(a0, b0, sg0)
        wout(nchunk - 1, a0, b0)

    return k(ei4, T1, T2)


def _edge_t_stats(e, G1, G2, wt, b, block):
    E, D = e.shape
    nb = E // block

    def body(e_ref, g1_ref, g2_ref, w_ref, b_ref, t_ref, st_ref, acc):
        i = pl.program_id(0)

        @pl.when(i == 0)
        def _():
            acc[...] = jnp.zeros_like(acc)

        t = (jnp.dot(e_ref[...], w_ref[...], precision=lax.Precision.HIGHEST,
                     preferred_element_type=F32)
             + b_ref[...] + g1_ref[...] + g2_ref[...])
        t_ref[...] = t
        t3 = t.reshape(block // 8, 8, D)
        acc[0] += jnp.sum(t3, axis=0)
        acc[1] += jnp.sum(t3 * t3, axis=0)

        @pl.when(i == nb - 1)
        def _():
            st_ref[...] = acc[...]

    blk = pl.BlockSpec((block, D), lambda i: (i, 0))
    return pl.pallas_call(
        body,
        grid=(nb,),
        in_specs=[blk, blk, blk,
                  pl.BlockSpec((D, D), lambda i: (0, 0)),
                  pl.BlockSpec((1, D), lambda i: (0, 0))],
        out_specs=[blk, pl.BlockSpec((2, 8, D), lambda i: (0, 0, 0))],
        out_shape=[jax.ShapeDtypeStruct((E, D), F32),
                   jax.ShapeDtypeStruct((2, 8, D), F32)],
        scratch_shapes=[pltpu.VMEM((2, 8, D), F32)],
    )(e, G1, G2, wt, b[None, :])


def _edge_sigma(t, e, G1, G2, stats, gamma, beta, block):
    E, D = e.shape
    nb = E // block

    def body(t_ref, e_ref, a2_ref, a3_ref, st_ref, g_ref, bt_ref, p_ref):
        mean = jnp.sum(st_ref[0], axis=0) / E
        var = jnp.sum(st_ref[1], axis=0) / E - mean * mean
        scale = g_ref[0] * lax.rsqrt(var + 1e-5)
        shift = bt_ref[0] - mean * scale
        bn = t_ref[...] * scale + shift
        ee = jnp.maximum(bn, 0.0) + e_ref[...]
        sg = jax.nn.sigmoid(ee)
        u2 = sg * a2_ref[...]
        u3 = sg * a3_ref[...]
        pieces = []
        for c in range(4):
            s = slice(DC * c, DC * (c + 1))
            pieces += [u2[:, s], sg[:, s], u3[:, s], sg[:, s]]
        p_ref[...] = jnp.concatenate(pieces, axis=1)

    blk = pl.BlockSpec((block, D), lambda i: (i, 0))
    blk1 = pl.BlockSpec((block, D), lambda i: (i, 1))
    small = pl.BlockSpec((1, D), lambda i: (0, 0))
    return pl.pallas_call(
        body,
        grid=(nb,),
        in_specs=[blk, blk, blk1, blk1,
                  pl.BlockSpec((2, 8, D), lambda i: (0, 0, 0)), small, small],
        out_specs=pl.BlockSpec((block, 4 * D), lambda i: (i, 0)),
        out_shape=jax.ShapeDtypeStruct((E, 4 * D), F32),
    )(t, e, G1, G2, stats, gamma[None, :], beta[None, :])


def _sc_scatter(ei4, P, zrows, N):
    E = P.shape[0]
    D = P.shape[1] // 4
    nchunk = E // (NW * K)
    npair = (nchunk - 1) // 2
    ncc = D // DC
    nrow = N // NS
    po = jax.ShapeDtypeStruct((NC, N, 2 * D), F32)

    @functools.partial(
        pl.kernel, mesh=_mesh,
        out_type=[po, po],
        compiler_params=pltpu.CompilerParams(use_tc_tiling_on_sc=False),
        scratch_types=[
            pltpu.VMEM((nchunk, K), jnp.int32),
            pltpu.VMEM((nchunk, K), jnp.int32),
            pltpu.VMEM((K, 2 * DC), F32),
            pltpu.VMEM((K, 2 * DC), F32),
            pltpu.VMEM((K, 2 * DC), F32),
            pltpu.VMEM((K, 2 * DC), F32),
            pltpu.VMEM_SHARED((N, 2 * DC), F32),
            pltpu.VMEM_SHARED((N, 2 * DC), F32),
            pltpu.SemaphoreType.DMA,
            pltpu.SemaphoreType.DMA,
            pltpu.SemaphoreType.DMA,
        ],
    )
    def k(ei_hbm, p_hbm, z_hbm, oFD, oBD,
          idxs2, idxd2, f0, b0, f1, b1, accFD, accBD, sr0, sr1, ss):
        cid = lax.axis_index("c")
        sid = lax.axis_index("s")
        wid = sid * NC + cid
        base = wid * nchunk
        r0 = sid * nrow
        pltpu.sync_copy(ei_hbm.at[0, wid], idxs2)
        pltpu.sync_copy(ei_hbm.at[1, wid], idxd2)
        for c in range(ncc):
            cs = c * 4 * DC
            pltpu.sync_copy(z_hbm, accFD.at[pl.ds(r0, nrow)])
            pltpu.sync_copy(z_hbm, accBD.at[pl.ds(r0, nrow)])
            plsc.subcore_barrier()

            def rd(i, bf, bb, sem):
                off = (base + i) * K
                pltpu.async_copy(
                    p_hbm.at[pl.ds(off, K), pl.ds(cs, 2 * DC)], bf, sem)
                pltpu.async_copy(
                    p_hbm.at[pl.ds(off, K), pl.ds(cs + 2 * DC, 2 * DC)], bb, sem)

            def wait_r(bf, bb, sem):
                pltpu.make_async_copy(
                    p_hbm.at[pl.ds(0, K), pl.ds(0, 2 * DC)], bf, sem).wait()
                pltpu.make_async_copy(
                    p_hbm.at[pl.ds(0, K), pl.ds(0, 2 * DC)], bb, sem).wait()

            def scat(i, bf, bb):
                c1 = pltpu.async_copy(bf, accFD.at[idxd2.at[i]], ss, add=True)
                c2 = pltpu.async_copy(bb, accBD.at[idxs2.at[i]], ss, add=True)
                c1.wait()
                c2.wait()

            rd(0, f0, b0, sr0)
            rd(1, f1, b1, sr1)

            @pl.loop(0, npair)
            def _(j):
                i = j * 2
                wait_r(f0, b0, sr0)
                scat(i, f0, b0)
                rd(i + 2, f0, b0, sr0)
                wait_r(f1, b1, sr1)
                scat(i + 1, f1, b1)

                @pl.when(j < npair - 1)
                def _():
                    rd(i + 3, f1, b1, sr1)

            wait_r(f0, b0, sr0)
            scat(nchunk - 1, f0, b0)

            plsc.subcore_barrier()
            pltpu.sync_copy(accFD.at[pl.ds(r0, nrow)],
                            oFD.at[cid, pl.ds(r0, nrow), pl.ds(c * 2 * DC, 2 * DC)])
            pltpu.sync_copy(accBD.at[pl.ds(r0, nrow)],
                            oBD.at[cid, pl.ds(r0, nrow), pl.ds(c * 2 * DC, 2 * DC)])
            plsc.subcore_barrier()

    return k(ei4, P, zrows)


def _final(A1h, h, oFD, oBD, gamma, beta, block):
    N, D = h.shape
    nb = N // block

    def body(a1_ref, h_ref, fd_ref, bd_ref, g_ref, bt_ref, o_ref, acc):
        p = pl.program_id(0)
        j = pl.program_id(1)
        hf, hb = [], []
        for c in range(4):
            a = slice(64 * c, 64 * c + 32)
            d = slice(64 * c + 32, 64 * c + 64)
            hf.append((fd_ref[0, :, a] + fd_ref[1, :, a])
                      / (fd_ref[0, :, d] + fd_ref[1, :, d] + 1e-6))
            hb.append((bd_ref[0, :, a] + bd_ref[1, :, a])
                      / (bd_ref[0, :, d] + bd_ref[1, :, d] + 1e-6))
        hp = (a1_ref[...] + jnp.concatenate(hf, axis=1)
              + jnp.concatenate(hb, axis=1))

        @pl.when(p == 0)
        def _():
            @pl.when(j == 0)
            def _():
                acc[...] = jnp.zeros_like(acc)

            h3 = hp.reshape(block // 8, 8, D)
            acc[0] += jnp.sum(h3, axis=0)
            acc[1] += jnp.sum(h3 * h3, axis=0)

        @pl.when(p == 1)
        def _():
            mean = jnp.sum(acc[0], axis=0) / N
            var = jnp.sum(acc[1], axis=0) / N - mean * mean
            scale = g_ref[0] * lax.rsqrt(var + 1e-5)
            shift = bt_ref[0] - mean * scale
            o_ref[...] = jnp.maximum(hp * scale + shift, 0.0) + h_ref[...]

    blk = pl.BlockSpec((block, D), lambda p, j: (j, 0))
    pblk = pl.BlockSpec((2, block, 2 * D), lambda p, j: (0, j, 0))
    small = pl.BlockSpec((1, D), lambda p, j: (0, 0))
    return pl.pallas_call(
        body,
        grid=(2, nb),
        in_specs=[blk, blk, pblk, pblk, small, small],
        out_specs=blk,
        out_shape=jax.ShapeDtypeStruct((N, D), F32),
        scratch_shapes=[pltpu.VMEM((2, 8, D), F32)],
    )(A1h, h, oFD, oBD, gamma[None, :], beta[None, :])


def kernel(edge_index, h, e, A1_W, A1_b, A2_W, A2_b, A3_W, A3_b,
           B1_W, B1_b, B2_W, B2_b, B3_W, B3_b,
           bn_h_gamma, bn_h_beta, bn_e_gamma, bn_e_beta):
    N, D = h.shape
    E = e.shape[0]
    nchunk = E // (NW * K)

    A1h, T1, T2 = _node_matmuls(
        h,
        (A1_W.T, B1_W.T, A2_W.T, B2_W.T, A3_W.T),
        (A1_b, B1_b, A2_b, B2_b, A3_b))

    ei4 = edge_index.reshape(2, NW, nchunk, K)
    G1, G2 = _sc_gather(ei4, T1, T2, E)

    t, stats = _edge_t_stats(e, G1, G2, B3_W.T, B3_b, 2000)
    P = _edge_sigma(t, e, G1, G2, stats, bn_e_gamma, bn_e_beta, 2000)

    zrows = jnp.zeros((N // NS, 2 * DC), F32)
    oFD, oBD = _sc_scatter(ei4, P, zrows, N)

    return _final(A1h, h, oFD, oBD, bn_h_gamma, bn_h_beta, 2000)


# speedup vs baseline: 3.6543x; 1.2861x over previous
import functools

import jax
import jax.numpy as jnp
from jax import lax
from jax.experimental import pallas as pl
from jax.experimental.pallas import tpu as pltpu
from jax.experimental.pallas import tpu_sc as plsc

F32 = jnp.float32
NC = 2
NS = 16
NW = NC * NS
K = 80
DC = 32

_mesh = plsc.VectorSubcoreMesh(core_axis_name="c", subcore_axis_name="s")


def _node_matmuls(h, wts, bs, block=2000):
    N, D = h.shape
    nb = N // block

    def body(h_ref, w_ref, b_ref, oa1, ot1, ot2):
        hh = h_ref[...]
        m = [jnp.dot(hh, w_ref[i], precision=lax.Precision.HIGHEST,
                     preferred_element_type=F32) + b_ref[i] for i in range(5)]
        oa1[...] = m[0]
        ot1[...] = jnp.concatenate([m[1], m[2]], axis=1)
        ot2[...] = jnp.concatenate([m[3], m[4]], axis=1)

    blk = pl.BlockSpec((block, D), lambda i: (i, 0))
    blk2 = pl.BlockSpec((block, 2 * D), lambda i: (i, 0))
    return pl.pallas_call(
        body,
        grid=(nb,),
        in_specs=[blk,
                  pl.BlockSpec((5, D, D), lambda i: (0, 0, 0)),
                  pl.BlockSpec((5, 1, D), lambda i: (0, 0, 0))],
        out_specs=[blk, blk2, blk2],
        out_shape=[jax.ShapeDtypeStruct((N, D), F32),
                   jax.ShapeDtypeStruct((N, 2 * D), F32),
                   jax.ShapeDtypeStruct((N, 2 * D), F32)],
    )(h, jnp.stack(wts), jnp.stack(bs)[:, None, :])


def _sc_gather(ei4, T1, T2, E):
    N, D2 = T1.shape
    nchunk = E // (NW * K)
    npair = (nchunk - 1) // 2
    fo = jax.ShapeDtypeStruct((E, D2), F32)

    @functools.partial(
        pl.kernel, mesh=_mesh,
        out_type=[fo, fo],
        scratch_types=[
            pltpu.VMEM((nchunk, K), jnp.int32),
            pltpu.VMEM((nchunk, K), jnp.int32),
            pltpu.VMEM((K, D2), F32),
            pltpu.VMEM((K, D2), F32),
            pltpu.VMEM((K, D2), F32),
            pltpu.VMEM((K, D2), F32),
            pltpu.SemaphoreType.DMA,
            pltpu.SemaphoreType.DMA,
            pltpu.SemaphoreType.DMA,
        ],
    )
    def k(ei_hbm, t1_hbm, t2_hbm, g1_hbm, g2_hbm,
          idxs2, idxd2, a0, b0, a1, b1, sg0, sg1, sw):
        wid = lax.axis_index("s") * NC + lax.axis_index("c")
        base = wid * nchunk
        pltpu.sync_copy(ei_hbm.at[0, wid], idxs2)
        pltpu.sync_copy(ei_hbm.at[1, wid], idxd2)

        def gath(i, bufa, bufb, sem):
            pltpu.async_copy(t1_hbm.at[idxs2.at[i]], bufa, sem)
            pltpu.async_copy(t2_hbm.at[idxd2.at[i]], bufb, sem)

        def wait_g(bufa, bufb, sem):
            pltpu.make_async_copy(g1_hbm.at[pl.ds(0, K)], bufa, sem).wait()
            pltpu.make_async_copy(g1_hbm.at[pl.ds(0, K)], bufb, sem).wait()

        def wout(i, bufa, bufb):
            off = (base + i) * K
            c1 = pltpu.async_copy(bufa, g1_hbm.at[pl.ds(off, K)], sw)
            c2 = pltpu.async_copy(bufb, g2_hbm.at[pl.ds(off, K)], sw)
            c1.wait()
            c2.wait()

        gath(0, a0, b0, sg0)
        gath(1, a1, b1, sg1)

        @pl.loop(0, npair)
        def _(j):
            i = j * 2
            wait_g(a0, b0, sg0)
            wout(i, a0, b0)
            gath(i + 2, a0, b0, sg0)
            wait_g(a1, b1, sg1)
            wout(i + 1, a1, b1)

            @pl.when(j < npair - 1)
            def _():
                gath(i + 3, a1, b1, sg1)

        wait_g(a0, b0, sg0)
        wout(nchunk - 1, a0, b0)

    return k(ei4, T1, T2)


def _edge_t_stats(e, G1, G2, wt, b, block):
    E, D = e.shape
    nb = E // block

    def body(e_ref, g1_ref, g2_ref, w_ref, b_ref, t_ref, st_ref, acc):
        i = pl.program_id(0)

        @pl.when(i == 0)
        def _():
            acc[...] = jnp.zeros_like(acc)

        t = (jnp.dot(e_ref[...], w_ref[...], precision=lax.Precision.HIGHEST,
                     preferred_element_type=F32)
             + b_ref[...] + g1_ref[...] + g2_ref[...])
        t_ref[...] = t
        t3 = t.reshape(block // 8, 8, D)
        acc[0] += jnp.sum(t3, axis=0)
        acc[1] += jnp.sum(t3 * t3, axis=0)

        @pl.when(i == nb - 1)
        def _():
            st_ref[...] = acc[...]

    blk = pl.BlockSpec((block, D), lambda i: (i, 0))
    return pl.pallas_call(
        body,
        grid=(nb,),
        in_specs=[blk, blk, blk,
                  pl.BlockSpec((D, D), lambda i: (0, 0)),
                  pl.BlockSpec((1, D), lambda i: (0, 0))],
        out_specs=[blk, pl.BlockSpec((2, 8, D), lambda i: (0, 0, 0))],
        out_shape=[jax.ShapeDtypeStruct((E, D), F32),
                   jax.ShapeDtypeStruct((2, 8, D), F32)],
        scratch_shapes=[pltpu.VMEM((2, 8, D), F32)],
    )(e, G1, G2, wt, b[None, :])


def _edge_sigma(t, e, G1, G2, stats, gamma, beta, block):
    E, D = e.shape
    nb = E // block

    def body(t_ref, e_ref, a2_ref, a3_ref, st_ref, g_ref, bt_ref,
             pf_ref, pb_ref):
        mean = jnp.sum(st_ref[0], axis=0) / E
        var = jnp.sum(st_ref[1], axis=0) / E - mean * mean
        scale = g_ref[0] * lax.rsqrt(var + 1e-5)
        shift = bt_ref[0] - mean * scale
        bn = t_ref[...] * scale + shift
        ee = jnp.maximum(bn, 0.0) + e_ref[...]
        sg = jax.nn.sigmoid(ee)
        u2 = sg * a2_ref[...]
        u3 = sg * a3_ref[...]

        def planes(u):
            p0 = jnp.concatenate([u[:, 0:32], sg[:, 0:32],
                                  u[:, 32:64], sg[:, 32:64]], axis=1)
            p1 = jnp.concatenate([u[:, 64:96], sg[:, 64:96],
                                  u[:, 96:128], sg[:, 96:128]], axis=1)
            return jnp.stack([p0, p1], axis=0)

        pf_ref[...] = planes(u2)
        pb_ref[...] = planes(u3)

    blk = pl.BlockSpec((block, D), lambda i: (i, 0))
    blk1 = pl.BlockSpec((block, D), lambda i: (i, 1))
    small = pl.BlockSpec((1, D), lambda i: (0, 0))
    return pl.pallas_call(
        body,
        grid=(nb,),
        in_specs=[blk, blk, blk1, blk1,
                  pl.BlockSpec((2, 8, D), lambda i: (0, 0, 0)), small, small],
        out_specs=[pl.BlockSpec((2, block, D), lambda i: (0, i, 0))] * 2,
        out_shape=[jax.ShapeDtypeStruct((2, E, D), F32)] * 2,
    )(t, e, G1, G2, stats, gamma[None, :], beta[None, :])


def _sc_scatter_dir(ei4, P, zrows, Np, idx_row):
    E = P.shape[1]
    D = P.shape[2]
    nchunk = E // (NW * K)
    npair = (nchunk - 1) // 2
    nrow = Np // NS
    po = jax.ShapeDtypeStruct((NC, 2, Np, D), F32)

    @functools.partial(
        pl.kernel, mesh=_mesh,
        out_type=po,
        scratch_types=[
            pltpu.VMEM((nchunk, K), jnp.int32),
            pltpu.VMEM((K, D), F32),
            pltpu.VMEM((K, D), F32),
            pltpu.VMEM_SHARED((Np, D), F32),
            pltpu.SemaphoreType.DMA,
            pltpu.SemaphoreType.DMA,
            pltpu.SemaphoreType.DMA,
        ],
    )
    def k(ei_hbm, p_hbm, z_hbm, out, idx2, f0, f1, acc, sr0, sr1, ss):
        cid = lax.axis_index("c")
        sid = lax.axis_index("s")
        wid = sid * NC + cid
        base = wid * nchunk
        r0 = sid * nrow
        pltpu.sync_copy(ei_hbm.at[idx_row, wid], idx2)
        for p in range(2):
            pltpu.sync_copy(z_hbm, acc.at[pl.ds(r0, nrow)])
            plsc.subcore_barrier()

            def rd(i, buf, sem):
                off = (base + i) * K
                pltpu.async_copy(p_hbm.at[p, pl.ds(off, K), :], buf, sem)

            def wait_r(buf, sem):
                pltpu.make_async_copy(
                    p_hbm.at[0, pl.ds(0, K), :], buf, sem).wait()

            def scat(i, buf):
                pltpu.async_copy(buf, acc.at[idx2.at[i]], ss, add=True).wait()

            rd(0, f0, sr0)
            rd(1, f1, sr1)

            @pl.loop(0, npair)
            def _(j):
                i = j * 2
                wait_r(f0, sr0)
                scat(i, f0)
                rd(i + 2, f0, sr0)
                wait_r(f1, sr1)
                scat(i + 1, f1)

                @pl.when(j < npair - 1)
                def _():
                    rd(i + 3, f1, sr1)

            wait_r(f0, sr0)
            scat(nchunk - 1, f0)

            plsc.subcore_barrier()
            pltpu.sync_copy(acc.at[pl.ds(r0, nrow)],
                            out.at[cid, p, pl.ds(r0, nrow), :])
            plsc.subcore_barrier()

    return k(ei4, P, zrows)


def _final(A1h, h, oFD, oBD, gamma, beta, block):
    N, D = h.shape
    nb = N // block

    def body(a1_ref, h_ref, fd_ref, bd_ref, g_ref, bt_ref, o_ref, acc):
        p = pl.program_id(0)
        j = pl.program_id(1)
        hf, hb = [], []
        for c in range(4):
            pp = c // 2
            a = slice(64 * (c % 2), 64 * (c % 2) + 32)
            d = slice(64 * (c % 2) + 32, 64 * (c % 2) + 64)
            hf.append((fd_ref[0, pp, :, a] + fd_ref[1, pp, :, a])
                      / (fd_ref[0, pp, :, d] + fd_ref[1, pp, :, d] + 1e-6))
            hb.append((bd_ref[0, pp, :, a] + bd_ref[1, pp, :, a])
                      / (bd_ref[0, pp, :, d] + bd_ref[1, pp, :, d] + 1e-6))
        hp = (a1_ref[...] + jnp.concatenate(hf, axis=1)
              + jnp.concatenate(hb, axis=1))

        @pl.when(p == 0)
        def _():
            @pl.when(j == 0)
            def _():
                acc[...] = jnp.zeros_like(acc)

            h3 = hp.reshape(block // 8, 8, D)
            acc[0] += jnp.sum(h3, axis=0)
            acc[1] += jnp.sum(h3 * h3, axis=0)

        @pl.when(p == 1)
        def _():
            mean = jnp.sum(acc[0], axis=0) / N
            var = jnp.sum(acc[1], axis=0) / N - mean * mean
            scale = g_ref[0] * lax.rsqrt(var + 1e-5)
            shift = bt_ref[0] - mean * scale
            o_ref[...] = jnp.maximum(hp * scale + shift, 0.0) + h_ref[...]

    blk = pl.BlockSpec((block, D), lambda p, j: (j, 0))
    pblk = pl.BlockSpec((2, 2, block, D), lambda p, j: (0, 0, j, 0))
    small = pl.BlockSpec((1, D), lambda p, j: (0, 0))
    return pl.pallas_call(
        body,
        grid=(2, nb),
        in_specs=[blk, blk, pblk, pblk, small, small],
        out_specs=blk,
        out_shape=jax.ShapeDtypeStruct((N, D), F32),
        scratch_shapes=[pltpu.VMEM((2, 8, D), F32)],
    )(A1h, h, oFD, oBD, gamma[None, :], beta[None, :])


def kernel(edge_index, h, e, A1_W, A1_b, A2_W, A2_b, A3_W, A3_b,
           B1_W, B1_b, B2_W, B2_b, B3_W, B3_b,
           bn_h_gamma, bn_h_beta, bn_e_gamma, bn_e_beta):
    N, D = h.shape
    E = e.shape[0]
    nchunk = E // (NW * K)

    A1h, T1, T2 = _node_matmuls(
        h,
        (A1_W.T, B1_W.T, A2_W.T, B2_W.T, A3_W.T),
        (A1_b, B1_b, A2_b, B2_b, A3_b))

    ei4 = edge_index.reshape(2, NW, nchunk, K)
    G1, G2 = _sc_gather(ei4, T1, T2, E)

    t, stats = _edge_t_stats(e, G1, G2, B3_W.T, B3_b, 2000)
    PF, PB = _edge_sigma(t, e, G1, G2, stats, bn_e_gamma, bn_e_beta, 2000)

    Np = ((N + 8 * NS - 1) // (8 * NS)) * (8 * NS)
    zrows = jnp.zeros((Np // NS, D), F32)
    oF = _sc_scatter_dir(ei4, PF, zrows, Np, 1)
    oB = _sc_scatter_dir(ei4, PB, zrows, Np, 0)

    return _final(A1h, h, oF, oB, bn_h_gamma, bn_h_beta, 2000)


# speedup vs baseline: 4.1446x; 1.1342x over previous
import functools

import jax
import jax.numpy as jnp
from jax import lax
from jax.experimental import pallas as pl
from jax.experimental.pallas import tpu as pltpu
from jax.experimental.pallas import tpu_sc as plsc

F32 = jnp.float32
NC = 2
NS = 16
NW = NC * NS
K = 80
DC = 32

_mesh = plsc.VectorSubcoreMesh(core_axis_name="c", subcore_axis_name="s")


def _unpack_hi(p):
    w = lax.bitcast_convert_type(p, jnp.uint32)
    return lax.bitcast_convert_type((w >> 16).astype(jnp.uint16),
                                    jnp.bfloat16).astype(F32)


def _unpack_lo(p):
    w = lax.bitcast_convert_type(p, jnp.uint32)
    return lax.bitcast_convert_type(w.astype(jnp.uint16),
                                    jnp.bfloat16).astype(F32)


def _node_matmuls(h, wts, bs, block=2000):
    N, D = h.shape
    nb = N // block

    def pack2(x, y):
        xu = lax.bitcast_convert_type(x.astype(jnp.bfloat16),
                                      jnp.uint16).astype(jnp.uint32)
        yu = lax.bitcast_convert_type(y.astype(jnp.bfloat16),
                                      jnp.uint16).astype(jnp.uint32)
        return lax.bitcast_convert_type((xu << 16) | yu, F32)

    def body(h_ref, w_ref, b_ref, oa1, ot1, ot2):
        hh = h_ref[...]
        m = [jnp.dot(hh, w_ref[i], precision=lax.Precision.HIGHEST,
                     preferred_element_type=F32) + b_ref[i] for i in range(5)]
        oa1[...] = m[0]
        ot1[...] = pack2(m[1], m[2])
        ot2[...] = pack2(m[3], m[4])

    blk = pl.BlockSpec((block, D), lambda i: (i, 0))
    return pl.pallas_call(
        body,
        grid=(nb,),
        in_specs=[blk,
                  pl.BlockSpec((5, D, D), lambda i: (0, 0, 0)),
                  pl.BlockSpec((5, 1, D), lambda i: (0, 0, 0))],
        out_specs=[blk, blk, blk],
        out_shape=[jax.ShapeDtypeStruct((N, D), F32)] * 3,
    )(h, jnp.stack(wts), jnp.stack(bs)[:, None, :])


def _sc_gather(ei4, T1, T2, E):
    N, D = T1.shape
    nchunk = E // (NW * K)
    npair = (nchunk - 1) // 2
    fo = jax.ShapeDtypeStruct((E, D), F32)

    @functools.partial(
        pl.kernel, mesh=_mesh,
        out_type=[fo, fo],
        scratch_types=[
            pltpu.VMEM((nchunk, K), jnp.int32),
            pltpu.VMEM((nchunk, K), jnp.int32),
            pltpu.VMEM((K, D), F32),
            pltpu.VMEM((K, D), F32),
            pltpu.VMEM((K, D), F32),
            pltpu.VMEM((K, D), F32),
            pltpu.SemaphoreType.DMA,
            pltpu.SemaphoreType.DMA,
            pltpu.SemaphoreType.DMA,
        ],
    )
    def k(ei_hbm, t1_hbm, t2_hbm, g1_hbm, g2_hbm,
          idxs2, idxd2, a0, b0, a1, b1, sg0, sg1, sw):
        wid = lax.axis_index("s") * NC + lax.axis_index("c")
        base = wid * nchunk
        pltpu.sync_copy(ei_hbm.at[0, wid], idxs2)
        pltpu.sync_copy(ei_hbm.at[1, wid], idxd2)

        def gath(i, bufa, bufb, sem):
            pltpu.async_copy(t1_hbm.at[idxs2.at[i]], bufa, sem)
            pltpu.async_copy(t2_hbm.at[idxd2.at[i]], bufb, sem)

        def wait_g(bufa, bufb, sem):
            pltpu.make_async_copy(g1_hbm.at[pl.ds(0, K)], bufa, sem).wait()
            pltpu.make_async_copy(g1_hbm.at[pl.ds(0, K)], bufb, sem).wait()

        def wout(i, bufa, bufb):
            off = (base + i) * K
            c1 = pltpu.async_copy(bufa, g1_hbm.at[pl.ds(off, K)], sw)
            c2 = pltpu.async_copy(bufb, g2_hbm.at[pl.ds(off, K)], sw)
            c1.wait()
            c2.wait()

        gath(0, a0, b0, sg0)
        gath(1, a1, b1, sg1)

        @pl.loop(0, npair)
        def _(j):
            i = j * 2
            wait_g(a0, b0, sg0)
            wout(i, a0, b0)
            gath(i + 2, a0, b0, sg0)
            wait_g(a1, b1, sg1)
            wout(i + 1, a1, b1)

            @pl.when(j < npair - 1)
            def _():
                gath(i + 3, a1, b1, sg1)

        wait_g(a0, b0, sg0)
        wout(nchunk - 1, a0, b0)

    return k(ei4, T1, T2)


def _edge_t_stats(e, G1, G2, wt, b, block):
    E, D = e.shape
    nb = E // block

    def body(e_ref, g1_ref, g2_ref, w_ref, b_ref, t_ref, st_ref, acc):
        i = pl.program_id(0)

        @pl.when(i == 0)
        def _():
            acc[...] = jnp.zeros_like(acc)

        t = (jnp.dot(e_ref[...], w_ref[...], precision=lax.Precision.HIGHEST,
                     preferred_element_type=F32)
             + b_ref[...] + _unpack_hi(g1_ref[...])
             + _unpack_hi(g2_ref[...]))
        t_ref[...] = t
        t3 = t.reshape(block // 8, 8, D)
        acc[0] += jnp.sum(t3, axis=0)
        acc[1] += jnp.sum(t3 * t3, axis=0)

        @pl.when(i == nb - 1)
        def _():
            st_ref[...] = acc[...]

    blk = pl.BlockSpec((block, D), lambda i: (i, 0))
    return pl.pallas_call(
        body,
        grid=(nb,),
        in_specs=[blk, blk, blk,
                  pl.BlockSpec((D, D), lambda i: (0, 0)),
                  pl.BlockSpec((1, D), lambda i: (0, 0))],
        out_specs=[blk, pl.BlockSpec((2, 8, D), lambda i: (0, 0, 0))],
        out_shape=[jax.ShapeDtypeStruct((E, D), F32),
                   jax.ShapeDtypeStruct((2, 8, D), F32)],
        scratch_shapes=[pltpu.VMEM((2, 8, D), F32)],
    )(e, G1, G2, wt, b[None, :])


def _edge_sigma(t, e, G1, G2, stats, gamma, beta, block):
    E, D = e.shape
    nb = E // block

    def body(t_ref, e_ref, a2_ref, a3_ref, st_ref, g_ref, bt_ref,
             pf_ref, pb_ref):
        mean = jnp.sum(st_ref[0], axis=0) / E
        var = jnp.sum(st_ref[1], axis=0) / E - mean * mean
        scale = g_ref[0] * lax.rsqrt(var + 1e-5)
        shift = bt_ref[0] - mean * scale
        bn = t_ref[...] * scale + shift
        ee = jnp.maximum(bn, 0.0) + e_ref[...]
        sg = jax.nn.sigmoid(ee)
        u2 = sg * _unpack_lo(a2_ref[...])
        u3 = sg * _unpack_lo(a3_ref[...])

        def planes(u):
            p0 = jnp.concatenate([u[:, 0:32], sg[:, 0:32],
                                  u[:, 32:64], sg[:, 32:64]], axis=1)
            p1 = jnp.concatenate([u[:, 64:96], sg[:, 64:96],
                                  u[:, 96:128], sg[:, 96:128]], axis=1)
            return jnp.stack([p0, p1], axis=0)

        pf_ref[...] = planes(u2)
        pb_ref[...] = planes(u3)

    blk = pl.BlockSpec((block, D), lambda i: (i, 0))
    small = pl.BlockSpec((1, D), lambda i: (0, 0))
    return pl.pallas_call(
        body,
        grid=(nb,),
        in_specs=[blk, blk, blk, blk,
                  pl.BlockSpec((2, 8, D), lambda i: (0, 0, 0)), small, small],
        out_specs=[pl.BlockSpec((2, block, D), lambda i: (0, i, 0))] * 2,
        out_shape=[jax.ShapeDtypeStruct((2, E, D), F32)] * 2,
    )(t, e, G1, G2, stats, gamma[None, :], beta[None, :])


def _sc_scatter_dir(ei4, P, zrows, Np, idx_row):
    E = P.shape[1]
    D = P.shape[2]
    nchunk = E // (NW * K)
    npair = (nchunk - 1) // 2
    nrow = Np // NS
    po = jax.ShapeDtypeStruct((NC, 2, Np, D), F32)

    @functools.partial(
        pl.kernel, mesh=_mesh,
        out_type=po,
        scratch_types=[
            pltpu.VMEM((nchunk, K), jnp.int32),
            pltpu.VMEM((K, D), F32),
            pltpu.VMEM((K, D), F32),
            pltpu.VMEM_SHARED((Np, D), F32),
            pltpu.SemaphoreType.DMA,
            pltpu.SemaphoreType.DMA,
            pltpu.SemaphoreType.DMA,
        ],
    )
    def k(ei_hbm, p_hbm, z_hbm, out, idx2, f0, f1, acc, sr0, sr1, ss):
        cid = lax.axis_index("c")
        sid = lax.axis_index("s")
        wid = sid * NC + cid
        base = wid * nchunk
        r0 = sid * nrow
        pltpu.sync_copy(ei_hbm.at[idx_row, wid], idx2)
        for p in range(2):
            pltpu.sync_copy(z_hbm, acc.at[pl.ds(r0, nrow)])
            plsc.subcore_barrier()

            def rd(i, buf, sem):
                off = (base + i) * K
                pltpu.async_copy(p_hbm.at[p, pl.ds(off, K), :], buf, sem)

            def wait_r(buf, sem):
                pltpu.make_async_copy(
                    p_hbm.at[0, pl.ds(0, K), :], buf, sem).wait()

            def scat(i, buf):
                pltpu.async_copy(buf, acc.at[idx2.at[i]], ss, add=True).wait()

            rd(0, f0, sr0)
            rd(1, f1, sr1)

            @pl.loop(0, npair)
            def _(j):
                i = j * 2
                wait_r(f0, sr0)
                scat(i, f0)
                rd(i + 2, f0, sr0)
                wait_r(f1, sr1)
                scat(i + 1, f1)

                @pl.when(j < npair - 1)
                def _():
                    rd(i + 3, f1, sr1)

            wait_r(f0, sr0)
            scat(nchunk - 1, f0)

            plsc.subcore_barrier()
            pltpu.sync_copy(acc.at[pl.ds(r0, nrow)],
                            out.at[cid, p, pl.ds(r0, nrow), :])
            plsc.subcore_barrier()

    return k(ei4, P, zrows)


def _final(A1h, h, oFD, oBD, gamma, beta, block):
    N, D = h.shape
    nb = N // block

    def body(a1_ref, h_ref, fd_ref, bd_ref, g_ref, bt_ref, o_ref, acc):
        p = pl.program_id(0)
        j = pl.program_id(1)
        hf, hb = [], []
        for c in range(4):
            pp = c // 2
            a = slice(64 * (c % 2), 64 * (c % 2) + 32)
            d = slice(64 * (c % 2) + 32, 64 * (c % 2) + 64)
            hf.append((fd_ref[0, pp, :, a] + fd_ref[1, pp, :, a])
                      / (fd_ref[0, pp, :, d] + fd_ref[1, pp, :, d] + 1e-6))
            hb.append((bd_ref[0, pp, :, a] + bd_ref[1, pp, :, a])
                      / (bd_ref[0, pp, :, d] + bd_ref[1, pp, :, d] + 1e-6))
        hp = (a1_ref[...] + jnp.concatenate(hf, axis=1)
              + jnp.concatenate(hb, axis=1))

        @pl.when(p == 0)
        def _():
            @pl.when(j == 0)
            def _():
                acc[...] = jnp.zeros_like(acc)

            h3 = hp.reshape(block // 8, 8, D)
            acc[0] += jnp.sum(h3, axis=0)
            acc[1] += jnp.sum(h3 * h3, axis=0)

        @pl.when(p == 1)
        def _():
            mean = jnp.sum(acc[0], axis=0) / N
            var = jnp.sum(acc[1], axis=0) / N - mean * mean
            scale = g_ref[0] * lax.rsqrt(var + 1e-5)
            shift = bt_ref[0] - mean * scale
            o_ref[...] = jnp.maximum(hp * scale + shift, 0.0) + h_ref[...]

    blk = pl.BlockSpec((block, D), lambda p, j: (j, 0))
    pblk = pl.BlockSpec((2, 2, block, D), lambda p, j: (0, 0, j, 0))
    small = pl.BlockSpec((1, D), lambda p, j: (0, 0))
    return pl.pallas_call(
        body,
        grid=(2, nb),
        in_specs=[blk, blk, pblk, pblk, small, small],
        out_specs=blk,
        out_shape=jax.ShapeDtypeStruct((N, D), F32),
        scratch_shapes=[pltpu.VMEM((2, 8, D), F32)],
    )(A1h, h, oFD, oBD, gamma[None, :], beta[None, :])


def kernel(edge_index, h, e, A1_W, A1_b, A2_W, A2_b, A3_W, A3_b,
           B1_W, B1_b, B2_W, B2_b, B3_W, B3_b,
           bn_h_gamma, bn_h_beta, bn_e_gamma, bn_e_beta):
    N, D = h.shape
    E = e.shape[0]
    nchunk = E // (NW * K)

    A1h, T1, T2 = _node_matmuls(
        h,
        (A1_W.T, B1_W.T, A2_W.T, B2_W.T, A3_W.T),
        (A1_b, B1_b, A2_b, B2_b, A3_b))

    ei4 = edge_index.reshape(2, NW, nchunk, K)
    G1, G2 = _sc_gather(ei4, T1, T2, E)

    t, stats = _edge_t_stats(e, G1, G2, B3_W.T, B3_b, 2000)
    PF, PB = _edge_sigma(t, e, G1, G2, stats, bn_e_gamma, bn_e_beta, 2000)

    Np = ((N + 8 * NS - 1) // (8 * NS)) * (8 * NS)
    zrows = jnp.zeros((Np // NS, D), F32)
    oF = _sc_scatter_dir(ei4, PF, zrows, Np, 1)
    oB = _sc_scatter_dir(ei4, PB, zrows, Np, 0)

    return _final(A1h, h, oF, oB, bn_h_gamma, bn_h_beta, 2000)
